# Initial kernel scaffold; baseline (speedup 1.0000x reference)
import jax
import jax.numpy as jnp
from jax.experimental import pallas as pl


def kernel(hidden_states, Wc, bc, Wi, Wo):
    raise NotImplementedError("write your pallas kernel here")


# speedup vs baseline: 2.3654x; 2.3654x over previous
import functools

import jax
import jax.numpy as jnp
from jax import lax
from jax.experimental import pallas as pl
from jax.experimental.pallas import tpu as pltpu
from jax.experimental.pallas import tpu_sc as plsc

N = 2048
D = 1024
FF = 2048
E = 8
CAP = 320
SLOTS = E * CAP
TRASH = SLOTS
XROWS = SLOTS + CAP
CHUNK = 128
NC = 2
NS = 16
NW = NC * NS
TPW = N // NW
FK = 2
FFC = FF // FK


def _router_body(x_ref, wc_ref, bc_ref,
                 probs_ref, top1_ref, ei_ref, fg_ref, fs_ref, scale_ref,
                 ndrop_ref, aux_ref):
    x = x_ref[...]
    wc = wc_ref[...]
    logits = lax.dot_general(wc, x, (((1,), (1,)), ((), ())),
                             preferred_element_type=jnp.float32)
    logits = logits + bc_ref[...]
    m = jnp.max(logits, axis=0, keepdims=True)
    ex = jnp.exp(logits - m)
    probs = ex / jnp.sum(ex, axis=0, keepdims=True)
    probs_ref[...] = probs
    top1 = jnp.max(probs, axis=0, keepdims=True)
    top1_ref[...] = top1

    row = lax.broadcasted_iota(jnp.int32, (E, N), 0)
    ai = jnp.min(jnp.where(probs == top1, row, E), axis=0, keepdims=True)
    onehot = (row == ai).astype(jnp.float32)

    ci = lax.broadcasted_iota(jnp.int32, (CHUNK, CHUNK), 0)
    cj = lax.broadcasted_iota(jnp.int32, (CHUNK, CHUNK), 1)
    tri = (ci <= cj).astype(jnp.float32)
    carry = jnp.zeros((E, 1), jnp.float32)
    pris = []
    for i in range(N // CHUNK):
        blk = onehot[:, i * CHUNK:(i + 1) * CHUNK]
        pris.append(carry + lax.dot(blk, tri,
                                    preferred_element_type=jnp.float32))
        carry = carry + jnp.sum(blk, axis=1, keepdims=True)
    pri = jnp.concatenate(pris, axis=1)

    mask = (pri <= float(CAP)).astype(jnp.float32)
    ei = onehot * mask
    ei_ref[...] = ei.astype(jnp.int32)
    kept = jnp.sum(ei, axis=0, keepdims=True) > 0.0
    pr_tok = jnp.sum(onehot * pri, axis=0, keepdims=True)
    slot = ai * CAP + pr_tok.astype(jnp.int32) - 1
    fg_ref[...] = jnp.where(kept, slot, 0)
    fs_ref[...] = jnp.where(kept, slot, TRASH)
    scale_ref[...] = jnp.where(kept, top1, 0.0)

    ndrop_ref[0, 0] = jnp.sum((~kept).astype(jnp.int32))
    fi = jnp.sum(ei, axis=1, keepdims=True) / float(N)
    pi = jnp.sum(probs, axis=1, keepdims=True) / float(N)
    aux_ref[0, 0] = float(E) * jnp.sum(fi * pi)


def _router(x, wc, bc_col):
    return pl.pallas_call(
        _router_body,
        out_shape=(
            jax.ShapeDtypeStruct((E, N), jnp.float32),
            jax.ShapeDtypeStruct((1, N), jnp.float32),
            jax.ShapeDtypeStruct((E, N), jnp.int32),
            jax.ShapeDtypeStruct((1, N), jnp.int32),
            jax.ShapeDtypeStruct((1, N), jnp.int32),
            jax.ShapeDtypeStruct((1, N), jnp.float32),
            jax.ShapeDtypeStruct((1, 1), jnp.int32),
            jax.ShapeDtypeStruct((1, 1), jnp.float32),
        ),
        out_specs=(
            pl.BlockSpec(memory_space=pltpu.VMEM),
            pl.BlockSpec(memory_space=pltpu.VMEM),
            pl.BlockSpec(memory_space=pltpu.VMEM),
            pl.BlockSpec(memory_space=pltpu.VMEM),
            pl.BlockSpec(memory_space=pltpu.VMEM),
            pl.BlockSpec(memory_space=pltpu.VMEM),
            pl.BlockSpec(memory_space=pltpu.SMEM),
            pl.BlockSpec(memory_space=pltpu.SMEM),
        ),
    )(x, wc, bc_col)


def _dispatch_body(x_hbm, fs_hbm, xbuf_hbm, idx_v, rows_v, sem):
    wid = lax.axis_index("s") * NC + lax.axis_index("c")
    base = wid * TPW
    pltpu.sync_copy(fs_hbm.at[pl.ds(base, TPW)], idx_v)
    pltpu.sync_copy(x_hbm.at[pl.ds(base, TPW)], rows_v)
    pltpu.async_copy(rows_v, xbuf_hbm.at[idx_v], sem).wait()


def _dispatch(x, fs):
    mesh = plsc.VectorSubcoreMesh(core_axis_name="c", subcore_axis_name="s")
    return pl.kernel(
        _dispatch_body,
        mesh=mesh,
        out_type=jax.ShapeDtypeStruct((XROWS, D), jnp.float32),
        scratch_types=[
            pltpu.VMEM((TPW,), jnp.int32),
            pltpu.VMEM((TPW, D), jnp.float32),
            pltpu.SemaphoreType.DMA,
        ],
    )(x, fs)


def _expert_body(x_ref, wi_ref, wo_ref, o_ref):
    k = pl.program_id(1)
    h = lax.dot_general(x_ref[...], wi_ref[0], (((1,), (1,)), ((), ())),
                        preferred_element_type=jnp.float32)
    h = jnp.maximum(h, 0.0)
    acc = lax.dot_general(h, wo_ref[0], (((1,), (1,)), ((), ())),
                          preferred_element_type=jnp.float32)

    @pl.when(k == 0)
    def _():
        o_ref[...] = acc

    @pl.when(k != 0)
    def _():
        o_ref[...] = o_ref[...] + acc


def _experts(xbuf, wi, wo):
    return pl.pallas_call(
        _expert_body,
        grid=(E, FK),
        in_specs=[
            pl.BlockSpec((CAP, D), lambda e, k: (e, 0)),
            pl.BlockSpec((1, FFC, D), lambda e, k: (e, k, 0)),
            pl.BlockSpec((1, D, FFC), lambda e, k: (e, 0, k)),
        ],
        out_specs=pl.BlockSpec((CAP, D), lambda e, k: (e, 0)),
        out_shape=jax.ShapeDtypeStruct((SLOTS, D), jnp.float32),
    )(xbuf, wi, wo)


def _combine_body(hbuf_hbm, fg_hbm, out_hbm, idx_v, rows_v, sem):
    wid = lax.axis_index("s") * NC + lax.axis_index("c")
    base = wid * TPW
    pltpu.sync_copy(fg_hbm.at[pl.ds(base, TPW)], idx_v)
    pltpu.async_copy(hbuf_hbm.at[idx_v], rows_v, sem).wait()
    pltpu.sync_copy(rows_v, out_hbm.at[pl.ds(base, TPW)])


def _combine(hbuf, fg):
    mesh = plsc.VectorSubcoreMesh(core_axis_name="c", subcore_axis_name="s")
    return pl.kernel(
        _combine_body,
        mesh=mesh,
        out_type=jax.ShapeDtypeStruct((N, D), jnp.float32),
        scratch_types=[
            pltpu.VMEM((TPW,), jnp.int32),
            pltpu.VMEM((TPW, D), jnp.float32),
            pltpu.SemaphoreType.DMA,
        ],
    )(hbuf, fg)


def _scale_body(g_ref, s_ref, o_ref):
    s = s_ref[...]
    o_ref[...] = jnp.where(s > 0.0, g_ref[...] * s, 0.0)


def _scale(g, s_col):
    rows = N // 8
    return pl.pallas_call(
        _scale_body,
        grid=(8,),
        in_specs=[
            pl.BlockSpec((rows, D), lambda i: (i, 0)),
            pl.BlockSpec((rows, 1), lambda i: (i, 0)),
        ],
        out_specs=pl.BlockSpec((rows, D), lambda i: (i, 0)),
        out_shape=jax.ShapeDtypeStruct((N, D), jnp.float32),
    )(g, s_col)


def kernel(hidden_states, Wc, bc, Wi, Wo):
    b, s, d = hidden_states.shape
    x = hidden_states.reshape(N, D)
    (probs_T, top1_T, ei_T, fg, fs, scale_T, ndrop, aux) = _router(
        x, Wc, bc.reshape(E, 1))
    xbuf = _dispatch(x, fs.reshape(N))
    hbuf = _experts(xbuf, Wi, Wo)
    g = _combine(hbuf, fg.reshape(N))
    out = _scale(g, scale_T.reshape(N, 1))
    return (out.reshape(b, s, d),
            ei_T.T,
            top1_T.reshape(N),
            probs_T.T,
            ndrop.reshape(()),
            aux.reshape(()))


# speedup vs baseline: 2.4068x; 1.0175x over previous
import functools

import jax
import jax.numpy as jnp
from jax import lax
from jax.experimental import pallas as pl
from jax.experimental.pallas import tpu as pltpu
from jax.experimental.pallas import tpu_sc as plsc

N = 2048
D = 1024
FF = 2048
E = 8
CAP = 320
SLOTS = E * CAP
TRASH = SLOTS
XROWS = SLOTS + CAP
CHUNK = 128
NC = 2
NS = 16
NW = NC * NS
TPW = N // NW
FK = 1
FFC = FF // FK


def _router_body(x_ref, wc_ref, bc_ref,
                 probs_ref, top1_ref, ei_ref, fg_ref, fs_ref, scale_ref,
                 ndrop_ref, aux_ref):
    x = x_ref[...]
    wc = wc_ref[...]
    logits = lax.dot_general(wc, x, (((1,), (1,)), ((), ())),
                             preferred_element_type=jnp.float32)
    logits = logits + bc_ref[...]
    m = jnp.max(logits, axis=0, keepdims=True)
    ex = jnp.exp(logits - m)
    probs = ex / jnp.sum(ex, axis=0, keepdims=True)
    probs_ref[...] = probs
    top1 = jnp.max(probs, axis=0, keepdims=True)
    top1_ref[...] = top1

    row = lax.broadcasted_iota(jnp.int32, (E, N), 0)
    ai = jnp.min(jnp.where(probs == top1, row, E), axis=0, keepdims=True)
    onehot = (row == ai).astype(jnp.float32)

    ci = lax.broadcasted_iota(jnp.int32, (CHUNK, CHUNK), 0)
    cj = lax.broadcasted_iota(jnp.int32, (CHUNK, CHUNK), 1)
    tri = (ci <= cj).astype(jnp.float32)
    carry = jnp.zeros((E, 1), jnp.float32)
    pris = []
    for i in range(N // CHUNK):
        blk = onehot[:, i * CHUNK:(i + 1) * CHUNK]
        pris.append(carry + lax.dot(blk, tri,
                                    preferred_element_type=jnp.float32))
        carry = carry + jnp.sum(blk, axis=1, keepdims=True)
    pri = jnp.concatenate(pris, axis=1)

    mask = (pri <= float(CAP)).astype(jnp.float32)
    ei = onehot * mask
    ei_ref[...] = ei.astype(jnp.int32)
    kept = jnp.sum(ei, axis=0, keepdims=True) > 0.0
    pr_tok = jnp.sum(onehot * pri, axis=0, keepdims=True)
    slot = ai * CAP + pr_tok.astype(jnp.int32) - 1
    fg_ref[...] = jnp.where(kept, slot, 0)
    fs_ref[...] = jnp.where(kept, slot, TRASH)
    scale_ref[...] = jnp.where(kept, top1, 0.0)

    ndrop_ref[0, 0] = jnp.sum((~kept).astype(jnp.int32))
    fi = jnp.sum(ei, axis=1, keepdims=True) / float(N)
    pi = jnp.sum(probs, axis=1, keepdims=True) / float(N)
    aux_ref[0, 0] = float(E) * jnp.sum(fi * pi)


def _router(x, wc, bc_col):
    return pl.pallas_call(
        _router_body,
        out_shape=(
            jax.ShapeDtypeStruct((E, N), jnp.float32),
            jax.ShapeDtypeStruct((1, N), jnp.float32),
            jax.ShapeDtypeStruct((E, N), jnp.int32),
            jax.ShapeDtypeStruct((1, N), jnp.int32),
            jax.ShapeDtypeStruct((1, N), jnp.int32),
            jax.ShapeDtypeStruct((1, N), jnp.float32),
            jax.ShapeDtypeStruct((1, 1), jnp.int32),
            jax.ShapeDtypeStruct((1, 1), jnp.float32),
        ),
        out_specs=(
            pl.BlockSpec(memory_space=pltpu.VMEM),
            pl.BlockSpec(memory_space=pltpu.VMEM),
            pl.BlockSpec(memory_space=pltpu.VMEM),
            pl.BlockSpec(memory_space=pltpu.VMEM),
            pl.BlockSpec(memory_space=pltpu.VMEM),
            pl.BlockSpec(memory_space=pltpu.VMEM),
            pl.BlockSpec(memory_space=pltpu.SMEM),
            pl.BlockSpec(memory_space=pltpu.SMEM),
        ),
    )(x, wc, bc_col)


def _dispatch_body(x_hbm, fs_hbm, xbuf_hbm, idx_v, rows_v, sem):
    wid = lax.axis_index("s") * NC + lax.axis_index("c")
    base = wid * TPW
    pltpu.sync_copy(fs_hbm.at[pl.ds(base, TPW)], idx_v)
    pltpu.sync_copy(x_hbm.at[pl.ds(base, TPW)], rows_v)
    pltpu.async_copy(rows_v, xbuf_hbm.at[idx_v], sem).wait()


def _dispatch(x, fs):
    mesh = plsc.VectorSubcoreMesh(core_axis_name="c", subcore_axis_name="s")
    return pl.kernel(
        _dispatch_body,
        mesh=mesh,
        out_type=jax.ShapeDtypeStruct((XROWS, D), jnp.float32),
        scratch_types=[
            pltpu.VMEM((TPW,), jnp.int32),
            pltpu.VMEM((TPW, D), jnp.float32),
            pltpu.SemaphoreType.DMA,
        ],
    )(x, fs)


def _expert_body(x_ref, wi_ref, wo_ref, o_ref):
    k = pl.program_id(1)
    x16 = x_ref[...].astype(jnp.bfloat16)
    h = lax.dot_general(x16, wi_ref[0].astype(jnp.bfloat16),
                        (((1,), (1,)), ((), ())),
                        preferred_element_type=jnp.float32)
    h16 = jnp.maximum(h, 0.0).astype(jnp.bfloat16)
    acc = lax.dot_general(h16, wo_ref[0].astype(jnp.bfloat16),
                          (((1,), (1,)), ((), ())),
                          preferred_element_type=jnp.float32)

    @pl.when(k == 0)
    def _():
        o_ref[...] = acc

    @pl.when(k != 0)
    def _():
        o_ref[...] = o_ref[...] + acc


def _experts(xbuf, wi, wo):
    return pl.pallas_call(
        _expert_body,
        grid=(E, FK),
        in_specs=[
            pl.BlockSpec((CAP, D), lambda e, k: (e, 0)),
            pl.BlockSpec((1, FFC, D), lambda e, k: (e, k, 0)),
            pl.BlockSpec((1, D, FFC), lambda e, k: (e, 0, k)),
        ],
        out_specs=pl.BlockSpec((CAP, D), lambda e, k: (e, 0)),
        out_shape=jax.ShapeDtypeStruct((SLOTS, D), jnp.float32),
    )(xbuf, wi, wo)


def _combine_body(hbuf_hbm, fg_hbm, out_hbm, idx_v, rows_v, sem):
    wid = lax.axis_index("s") * NC + lax.axis_index("c")
    base = wid * TPW
    pltpu.sync_copy(fg_hbm.at[pl.ds(base, TPW)], idx_v)
    pltpu.async_copy(hbuf_hbm.at[idx_v], rows_v, sem).wait()
    pltpu.sync_copy(rows_v, out_hbm.at[pl.ds(base, TPW)])


def _combine(hbuf, fg):
    mesh = plsc.VectorSubcoreMesh(core_axis_name="c", subcore_axis_name="s")
    return pl.kernel(
        _combine_body,
        mesh=mesh,
        out_type=jax.ShapeDtypeStruct((N, D), jnp.float32),
        scratch_types=[
            pltpu.VMEM((TPW,), jnp.int32),
            pltpu.VMEM((TPW, D), jnp.float32),
            pltpu.SemaphoreType.DMA,
        ],
    )(hbuf, fg)


def _scale_body(g_ref, s_ref, o_ref):
    s = s_ref[...]
    o_ref[...] = jnp.where(s > 0.0, g_ref[...] * s, 0.0)


def _scale(g, s_col):
    rows = N // 8
    return pl.pallas_call(
        _scale_body,
        grid=(8,),
        in_specs=[
            pl.BlockSpec((rows, D), lambda i: (i, 0)),
            pl.BlockSpec((rows, 1), lambda i: (i, 0)),
        ],
        out_specs=pl.BlockSpec((rows, D), lambda i: (i, 0)),
        out_shape=jax.ShapeDtypeStruct((N, D), jnp.float32),
    )(g, s_col)


def kernel(hidden_states, Wc, bc, Wi, Wo):
    b, s, d = hidden_states.shape
    x = hidden_states.reshape(N, D)
    (probs_T, top1_T, ei_T, fg, fs, scale_T, ndrop, aux) = _router(
        x, Wc, bc.reshape(E, 1))
    xbuf = _dispatch(x, fs.reshape(N))
    hbuf = _experts(xbuf, Wi, Wo)
    g = _combine(hbuf, fg.reshape(N))
    out = _scale(g, scale_T.reshape(N, 1))
    return (out.reshape(b, s, d),
            ei_T.T,
            top1_T.reshape(N),
            probs_T.T,
            ndrop.reshape(()),
            aux.reshape(()))


# speedup vs baseline: 2.5905x; 1.0763x over previous
import functools

import jax
import jax.numpy as jnp
from jax import lax
from jax.experimental import pallas as pl
from jax.experimental.pallas import tpu as pltpu
from jax.experimental.pallas import tpu_sc as plsc

N = 2048
D = 1024
FF = 2048
E = 8
CAP = 320
SLOTS = E * CAP
TRASH = SLOTS
XROWS = SLOTS + CAP
CHUNK = 128
NC = 2
NS = 16
NW = NC * NS
TPW = N // NW
FK = 1
FFC = FF // FK


def _router_body(x_ref, wc_ref, bc_ref,
                 probs_ref, top1_ref, ei_ref, fg_ref, fs_ref, xs_ref,
                 rowmask_ref, ndrop_ref, aux_ref):
    x = x_ref[...]
    wc = wc_ref[...]
    logits = lax.dot_general(wc, x, (((1,), (1,)), ((), ())),
                             preferred_element_type=jnp.float32)
    logits = logits + bc_ref[...]
    m = jnp.max(logits, axis=0, keepdims=True)
    ex = jnp.exp(logits - m)
    probs = ex / jnp.sum(ex, axis=0, keepdims=True)
    probs_ref[...] = probs
    top1 = jnp.max(probs, axis=0, keepdims=True)
    top1_ref[...] = top1

    row = lax.broadcasted_iota(jnp.int32, (E, N), 0)
    ai = jnp.min(jnp.where(probs == top1, row, E), axis=0, keepdims=True)
    onehot = (row == ai).astype(jnp.float32)

    ci = lax.broadcasted_iota(jnp.int32, (CHUNK, CHUNK), 0)
    cj = lax.broadcasted_iota(jnp.int32, (CHUNK, CHUNK), 1)
    tri = (ci <= cj).astype(jnp.float32)
    carry = jnp.zeros((E, 1), jnp.float32)
    pris = []
    for i in range(N // CHUNK):
        blk = onehot[:, i * CHUNK:(i + 1) * CHUNK]
        pris.append(carry + lax.dot(blk, tri,
                                    preferred_element_type=jnp.float32))
        carry = carry + jnp.sum(blk, axis=1, keepdims=True)
    pri = jnp.concatenate(pris, axis=1)

    mask = (pri <= float(CAP)).astype(jnp.float32)
    ei = onehot * mask
    ei_ref[...] = ei.astype(jnp.int32)
    kept = jnp.sum(ei, axis=0, keepdims=True) > 0.0
    pr_tok = jnp.sum(onehot * pri, axis=0, keepdims=True)
    slot = ai * CAP + pr_tok.astype(jnp.int32) - 1

    cnt8 = jnp.sum(ei, axis=1, keepdims=True).astype(jnp.int32)
    e_iota = lax.broadcasted_iota(jnp.int32, (E, 1), 0)
    mn = jnp.min(cnt8, axis=0, keepdims=True)
    am = jnp.min(jnp.where(cnt8 == mn, e_iota, E), axis=0, keepdims=True)
    zslot = am * CAP + mn
    fg_ref[...] = jnp.where(kept, slot, zslot)
    fs_ref[...] = jnp.where(kept, slot, TRASH)
    s_iota = lax.broadcasted_iota(jnp.int32, (E, CAP), 1)
    rowmask_ref[...] = (s_iota < cnt8).astype(jnp.float32)

    scale = jnp.where(kept, top1, 0.0)
    ident = (ci == cj).astype(jnp.float32)
    cols = []
    for i in range(N // CHUNK):
        blk = scale[:, i * CHUNK:(i + 1) * CHUNK]
        cols.append(lax.dot_general(ident, blk, (((1,), (1,)), ((), ())),
                                    preferred_element_type=jnp.float32))
    scale_col = jnp.concatenate(cols, axis=0)
    xs_ref[...] = x * scale_col

    ndrop_ref[0, 0] = jnp.sum((~kept).astype(jnp.int32))
    fi = jnp.sum(ei, axis=1, keepdims=True) / float(N)
    pi = jnp.sum(probs, axis=1, keepdims=True) / float(N)
    aux_ref[0, 0] = float(E) * jnp.sum(fi * pi)


def _router(x, wc, bc_col):
    return pl.pallas_call(
        _router_body,
        out_shape=(
            jax.ShapeDtypeStruct((E, N), jnp.float32),
            jax.ShapeDtypeStruct((1, N), jnp.float32),
            jax.ShapeDtypeStruct((E, N), jnp.int32),
            jax.ShapeDtypeStruct((1, N), jnp.int32),
            jax.ShapeDtypeStruct((1, N), jnp.int32),
            jax.ShapeDtypeStruct((N, D), jnp.float32),
            jax.ShapeDtypeStruct((E, CAP), jnp.float32),
            jax.ShapeDtypeStruct((1, 1), jnp.int32),
            jax.ShapeDtypeStruct((1, 1), jnp.float32),
        ),
        out_specs=(
            pl.BlockSpec(memory_space=pltpu.VMEM),
            pl.BlockSpec(memory_space=pltpu.VMEM),
            pl.BlockSpec(memory_space=pltpu.VMEM),
            pl.BlockSpec(memory_space=pltpu.VMEM),
            pl.BlockSpec(memory_space=pltpu.VMEM),
            pl.BlockSpec(memory_space=pltpu.VMEM),
            pl.BlockSpec(memory_space=pltpu.VMEM),
            pl.BlockSpec(memory_space=pltpu.SMEM),
            pl.BlockSpec(memory_space=pltpu.SMEM),
        ),
    )(x, wc, bc_col)


def _dispatch_body(x_hbm, fs_hbm, xbuf_hbm, idx_v, rows_v, sem):
    wid = lax.axis_index("s") * NC + lax.axis_index("c")
    base = wid * TPW
    pltpu.sync_copy(fs_hbm.at[pl.ds(base, TPW)], idx_v)
    pltpu.sync_copy(x_hbm.at[pl.ds(base, TPW)], rows_v)
    pltpu.async_copy(rows_v, xbuf_hbm.at[idx_v], sem).wait()


def _dispatch(x, fs):
    mesh = plsc.VectorSubcoreMesh(core_axis_name="c", subcore_axis_name="s")
    return pl.kernel(
        _dispatch_body,
        mesh=mesh,
        out_type=jax.ShapeDtypeStruct((XROWS, D), jnp.float32),
        scratch_types=[
            pltpu.VMEM((TPW,), jnp.int32),
            pltpu.VMEM((TPW, D), jnp.float32),
            pltpu.SemaphoreType.DMA,
        ],
    )(x, fs)


def _expert_body(x_ref, wi_ref, wo_ref, m_ref, o_ref):
    k = pl.program_id(1)
    x16 = x_ref[...].astype(jnp.bfloat16)
    h = lax.dot_general(x16, wi_ref[0].astype(jnp.bfloat16),
                        (((1,), (1,)), ((), ())),
                        preferred_element_type=jnp.float32)
    h16 = jnp.maximum(h, 0.0).astype(jnp.bfloat16)
    acc = lax.dot_general(h16, wo_ref[0].astype(jnp.bfloat16),
                          (((1,), (1,)), ((), ())),
                          preferred_element_type=jnp.float32)
    acc = jnp.where(m_ref[...] > 0.0, acc, 0.0)

    @pl.when(k == 0)
    def _():
        o_ref[...] = acc

    @pl.when(k != 0)
    def _():
        o_ref[...] = o_ref[...] + acc


def _experts(xbuf, wi, wo, rowmask_col):
    return pl.pallas_call(
        _expert_body,
        grid=(E, FK),
        in_specs=[
            pl.BlockSpec((CAP, D), lambda e, k: (e, 0)),
            pl.BlockSpec((1, FFC, D), lambda e, k: (e, k, 0)),
            pl.BlockSpec((1, D, FFC), lambda e, k: (e, 0, k)),
            pl.BlockSpec((CAP, 1), lambda e, k: (e, 0)),
        ],
        out_specs=pl.BlockSpec((CAP, D), lambda e, k: (e, 0)),
        out_shape=jax.ShapeDtypeStruct((SLOTS, D), jnp.float32),
    )(xbuf, wi, wo, rowmask_col)


def _combine_body(hbuf_hbm, fg_hbm, out_hbm, idx_v, rows_v, sem):
    wid = lax.axis_index("s") * NC + lax.axis_index("c")
    base = wid * TPW
    pltpu.sync_copy(fg_hbm.at[pl.ds(base, TPW)], idx_v)
    pltpu.async_copy(hbuf_hbm.at[idx_v], rows_v, sem).wait()
    pltpu.sync_copy(rows_v, out_hbm.at[pl.ds(base, TPW)])


def _combine(hbuf, fg):
    mesh = plsc.VectorSubcoreMesh(core_axis_name="c", subcore_axis_name="s")
    return pl.kernel(
        _combine_body,
        mesh=mesh,
        out_type=jax.ShapeDtypeStruct((N, D), jnp.float32),
        scratch_types=[
            pltpu.VMEM((TPW,), jnp.int32),
            pltpu.VMEM((TPW, D), jnp.float32),
            pltpu.SemaphoreType.DMA,
        ],
    )(hbuf, fg)


def kernel(hidden_states, Wc, bc, Wi, Wo):
    b, s, d = hidden_states.shape
    x = hidden_states.reshape(N, D)
    (probs_T, top1_T, ei_T, fg, fs, xs, rowmask, ndrop, aux) = _router(
        x, Wc, bc.reshape(E, 1))
    xbuf = _dispatch(xs, fs.reshape(N))
    hbuf = _experts(xbuf, Wi, Wo, rowmask.reshape(SLOTS, 1))
    out = _combine(hbuf, fg.reshape(N))
    return (out.reshape(b, s, d),
            ei_T.T,
            top1_T.reshape(N),
            probs_T.T,
            ndrop.reshape(()),
            aux.reshape(()))
